# baseline (speedup 1.0000x reference)
import functools

import jax
import jax.numpy as jnp
from jax import lax
from jax.experimental import pallas as pl
from jax.experimental.pallas import tpu as pltpu
from jax.experimental.pallas import tpu_sc as plsc

_B, _L, _E = 1024, 200, 64
_MAXLEN = 512
_BPC = 2
_CR = _BPC * _L


@functools.cache
def _make_sc_kernel():
    info = plsc.get_sparse_core_info()
    nc, ns = info.num_cores, info.num_subcores
    nw = nc * ns
    bpw = _B // nw
    rpw = bpw * _L
    nch = bpw // _BPC
    nbuf = 3
    mesh = plsc.VectorSubcoreMesh(core_axis_name="c", subcore_axis_name="s",
                                  num_cores=nc)

    @functools.partial(
        pl.kernel,
        mesh=mesh,
        compiler_params=pltpu.CompilerParams(needs_layout_passes=False,
                                             use_tc_tiling_on_sc=False),
        out_type=jax.ShapeDtypeStruct((_B, _L, _E), jnp.float32),
        scratch_types=[
            pltpu.VMEM((_L, _E), jnp.float32),
            pltpu.VMEM((2, _E), jnp.float32),
            pltpu.VMEM((2 * _L * _E,), jnp.float32),
            pltpu.VMEM((rpw,), jnp.int32),
            pltpu.VMEM((rpw,), jnp.int32),
            pltpu.VMEM((_CR, _E), jnp.float32),
            pltpu.VMEM((_CR, _E), jnp.float32),
            pltpu.VMEM((_CR, _E), jnp.float32),
            pltpu.SemaphoreType.DMA,
            pltpu.SemaphoreType.DMA,
            pltpu.SemaphoreType.DMA,
            pltpu.SemaphoreType.DMA,
            pltpu.SemaphoreType.DMA,
            pltpu.SemaphoreType.DMA,
            pltpu.SemaphoreType.DMA,
            pltpu.SemaphoreType.DMA,
        ],
    )
    def sc_kernel(ids_hbm, segids_hbm, tok_hbm, pos_hbm, seg_hbm, out_hbm,
                  pos_v, seg_v, comb_v, ids_all, seg_all, rows0, rows1, rows2,
                  sem_i, sem_sg, sg0, sg1, sg2, so0, so1, so2):
        rows_b = (rows0, rows1, rows2)
        sem_g = (sg0, sg1, sg2)
        sem_o = (so0, so1, so2)

        wid = lax.axis_index("s") * nc + lax.axis_index("c")
        bbase = wid * bpw

        for b in range(bpw):
            pltpu.async_copy(ids_hbm.at[bbase + b],
                             ids_all.at[pl.ds(b * _L, _L)], sem_i)
            pltpu.async_copy(segids_hbm.at[bbase + b],
                             seg_all.at[pl.ds(b * _L, _L)], sem_sg)

        pltpu.sync_copy(pos_hbm.at[pl.ds(0, _L)], pos_v)
        pltpu.sync_copy(seg_hbm.at[pl.ds(0, 2)], seg_v)

        def build(l, carry):
            for j in range(_E // 16):
                p = pos_v[l, pl.ds(16 * j, 16)]
                s0 = seg_v[0, pl.ds(16 * j, 16)]
                s1 = seg_v[1, pl.ds(16 * j, 16)]
                comb_v[pl.ds(2 * l * _E + 16 * j, 16)] = p + s0
                comb_v[pl.ds((2 * l + 1) * _E + 16 * j, 16)] = p + s1
            return carry

        lax.fori_loop(0, _L, build, 0, unroll=False)

        for b in range(bpw):
            pltpu.make_async_copy(ids_hbm.at[0],
                                  ids_all.at[pl.ds(b * _L, _L)], sem_i).wait()
            pltpu.make_async_copy(segids_hbm.at[0],
                                  seg_all.at[pl.ds(b * _L, _L)], sem_sg).wait()

        def enq_gather(k, p):
            pltpu.async_copy(
                tok_hbm.at[ids_all.at[pl.ds(k * _CR, _CR)]],
                rows_b[p], sem_g[p])

        def wait_gather(p):
            pltpu.make_async_copy(tok_hbm.at[ids_all.at[pl.ds(0, _CR)]],
                                  rows_b[p], sem_g[p]).wait()

        def enq_out(k, p):
            b = bbase + _BPC * k
            for i in range(_BPC):
                pltpu.async_copy(rows_b[p].at[pl.ds(i * _L, _L)],
                                 out_hbm.at[b + i], sem_o[p])

        def wait_out(p):
            for i in range(_BPC):
                pltpu.make_async_copy(rows_b[p].at[pl.ds(i * _L, _L)],
                                      out_hbm.at[0], sem_o[p]).wait()

        lane = lax.iota(jnp.int32, 16)
        zeros = lane - lane

        def compute(k, p):
            rows = rows_b[p]

            def group(g, carry):
                row_vec = g * 16 + lane
                l_vec = lax.rem(row_vec, _L)
                seg_vec = seg_all[pl.ds(k * _CR + g * 16, 16)]
                cvec = (l_vec * 2 + seg_vec) * _E
                for c in range(_E):
                    a = plsc.load_gather(comb_v, [cvec + c])
                    plsc.addupdate_scatter(rows, [row_vec, zeros + c], a)
                return carry

            lax.fori_loop(0, _CR // 16, group, 0, unroll=False)

        enq_gather(0, 0)
        enq_gather(1, 1)

        def chunk_body(k, p, i=None):
            wait_gather(p)
            if i is None:
                pass
            else:
                @pl.when((k >= 1) & (k + 2 < nch))
                def _():
                    wait_out((p + 2) % nbuf)
                    enq_gather(k + 2, (p + 2) % nbuf)

                @pl.when((k < 1) & (k + 2 < nch))
                def _():
                    enq_gather(k + 2, (p + 2) % nbuf)
            compute(k, p)
            enq_out(k, p)

        def triple(i, carry):
            for c in range(3):
                chunk_body(3 * i + c, c, i)
            return carry

        lax.fori_loop(0, nch // nbuf, triple, 0, unroll=False)
        chunk_body(nch - 1, (nch - 1) % nbuf)
        wait_out((nch - 3) % nbuf)
        wait_out((nch - 2) % nbuf)
        wait_out((nch - 1) % nbuf)

    return sc_kernel


def kernel(input_ids, segment_ids, token_table, pos_table, seg_table):
    return _make_sc_kernel()(input_ids, segment_ids, token_table, pos_table,
                             seg_table)


# speedup vs baseline: 1.5347x; 1.5347x over previous
import functools

import jax
import jax.numpy as jnp
from jax import lax
from jax.experimental import pallas as pl
from jax.experimental.pallas import tpu as pltpu
from jax.experimental.pallas import tpu_sc as plsc

_B, _L, _E = 1024, 200, 64
_BPC = 2
_CR = _BPC * _L


@functools.cache
def _make_sc_gather():
    info = plsc.get_sparse_core_info()
    nc, ns = info.num_cores, info.num_subcores
    nw = nc * ns
    bpw = _B // nw
    rpw = bpw * _L
    nch = bpw // _BPC
    nbuf = 3
    mesh = plsc.VectorSubcoreMesh(core_axis_name="c", subcore_axis_name="s",
                                  num_cores=nc)

    @functools.partial(
        pl.kernel,
        mesh=mesh,
        compiler_params=pltpu.CompilerParams(needs_layout_passes=False,
                                             use_tc_tiling_on_sc=False),
        out_type=jax.ShapeDtypeStruct((_B, _L, _E), jnp.float32),
        scratch_types=[
            pltpu.VMEM((rpw,), jnp.int32),
            pltpu.VMEM((_CR, _E), jnp.float32),
            pltpu.VMEM((_CR, _E), jnp.float32),
            pltpu.VMEM((_CR, _E), jnp.float32),
            pltpu.SemaphoreType.DMA,
            pltpu.SemaphoreType.DMA,
            pltpu.SemaphoreType.DMA,
            pltpu.SemaphoreType.DMA,
            pltpu.SemaphoreType.DMA,
            pltpu.SemaphoreType.DMA,
            pltpu.SemaphoreType.DMA,
        ],
    )
    def sc_gather(ids_hbm, tok_hbm, out_hbm,
                  ids_all, rows0, rows1, rows2,
                  sem_i, sg0, sg1, sg2, so0, so1, so2):
        rows_b = (rows0, rows1, rows2)
        sem_g = (sg0, sg1, sg2)
        sem_o = (so0, so1, so2)

        wid = lax.axis_index("s") * nc + lax.axis_index("c")
        bbase = wid * bpw

        for b in range(bpw):
            pltpu.async_copy(ids_hbm.at[bbase + b],
                             ids_all.at[pl.ds(b * _L, _L)], sem_i)
        for b in range(bpw):
            pltpu.make_async_copy(ids_hbm.at[0],
                                  ids_all.at[pl.ds(b * _L, _L)], sem_i).wait()

        def enq_gather(k, p):
            pltpu.async_copy(tok_hbm.at[ids_all.at[pl.ds(k * _CR, _CR)]],
                             rows_b[p], sem_g[p])

        def wait_gather(p):
            pltpu.make_async_copy(tok_hbm.at[ids_all.at[pl.ds(0, _CR)]],
                                  rows_b[p], sem_g[p]).wait()

        def enq_out(k, p):
            b = bbase + _BPC * k
            for i in range(_BPC):
                pltpu.async_copy(rows_b[p].at[pl.ds(i * _L, _L)],
                                 out_hbm.at[b + i], sem_o[p])

        def wait_out(p):
            for i in range(_BPC):
                pltpu.make_async_copy(rows_b[p].at[pl.ds(i * _L, _L)],
                                      out_hbm.at[0], sem_o[p]).wait()

        enq_gather(0, 0)
        enq_gather(1, 1)

        def chunk_body(k, p, prefetch):
            wait_gather(p)
            if prefetch:
                @pl.when((k >= 1) & (k + 2 < nch))
                def _():
                    wait_out((p + 2) % nbuf)
                    enq_gather(k + 2, (p + 2) % nbuf)

                @pl.when((k < 1) & (k + 2 < nch))
                def _():
                    enq_gather(k + 2, (p + 2) % nbuf)
            enq_out(k, p)

        def triple(i, carry):
            for c in range(3):
                chunk_body(3 * i + c, c, True)
            return carry

        lax.fori_loop(0, nch // nbuf, triple, 0, unroll=False)
        chunk_body(nch - 1, (nch - 1) % nbuf, False)
        wait_out((nch - 3) % nbuf)
        wait_out((nch - 2) % nbuf)
        wait_out((nch - 1) % nbuf)

    return sc_gather


def kernel(input_ids, segment_ids, token_table, pos_table, seg_table):
    tok = _make_sc_gather()(input_ids, token_table)
    seg = jnp.where(segment_ids[:, :, None] == 0, seg_table[0], seg_table[1])
    return tok + pos_table[None, :_L, :] + seg
